# baseline (speedup 1.0000x reference)
import functools

import jax
import jax.numpy as jnp
from jax import lax
from jax.experimental import pallas as pl
from jax.experimental.pallas import tpu as pltpu
from jax.experimental.pallas import tpu_sc as plsc

NC = 2
NS = 16
NW = NC * NS
L = 16

BATCH = 4096
SEQ = 200
HIDDEN = 64
NJ = HIDDEN // L

ROWS_PER_W = BATCH // NW
CHUNK = SEQ
NCHUNKS = ROWS_PER_W
TOK_PER_W = ROWS_PER_W * SEQ

_mesh = plsc.VectorSubcoreMesh(
    core_axis_name="c", subcore_axis_name="s", num_cores=NC, num_subcores=NS
)

_TAKE_DNUMS = lax.GatherDimensionNumbers(
    offset_dims=(), collapsed_slice_dims=(0,), start_index_map=(0,)
)


def _splat(vec, lane):
    idx = jnp.broadcast_to(lane, (L,)).astype(jnp.int32)
    return lax.gather(vec, idx[:, None], _TAKE_DNUMS, (1,),
                      mode=lax.GatherScatterMode.PROMISE_IN_BOUNDS)


@functools.partial(
    pl.kernel,
    out_type=(
        jax.ShapeDtypeStruct((BATCH * SEQ, HIDDEN), jnp.float32),
        jax.ShapeDtypeStruct((BATCH, HIDDEN), jnp.float32),
    ),
    mesh=_mesh,
    compiler_params=pltpu.CompilerParams(use_tc_tiling_on_sc=False,
                                         needs_layout_passes=False),
    scratch_types=[
        pltpu.VMEM((2, CHUNK), jnp.int32),
        pltpu.VMEM((2, CHUNK), jnp.int32),
        pltpu.VMEM((2, CHUNK, HIDDEN), jnp.float32),
        pltpu.VMEM((2, CHUNK, HIDDEN), jnp.float32),
        pltpu.VMEM((SEQ * HIDDEN,), jnp.float32),
        pltpu.VMEM((ROWS_PER_W, HIDDEN), jnp.float32),
        pltpu.SemaphoreType.DMA,
        pltpu.SemaphoreType.DMA,
        pltpu.SemaphoreType.DMA,
        pltpu.SemaphoreType.DMA,
    ],
)
def _encoder_sc(ids_hbm, pos_hbm, emb_hbm, pot_hbm, summed_hbm, pooled_hbm,
                ids_v, pos_v, e_buf, s_buf, pot_v, pool_buf,
                g_sem0, g_sem1, o_sem0, o_sem1):
    wid = lax.axis_index("s") * NC + lax.axis_index("c")
    w_base = wid * TOK_PER_W
    inv_seq = jnp.float32(1.0 / SEQ)
    g_sems = (g_sem0, g_sem1)
    o_sems = (o_sem0, o_sem1)
    iota = lax.iota(jnp.int32, L)

    pltpu.sync_copy(pot_hbm, pot_v)

    def tok0_of(c):
        return pl.multiple_of(w_base + c * CHUNK, CHUNK)

    def fetch(c, slot):
        tok0 = tok0_of(c)
        pltpu.sync_copy(ids_hbm.at[pl.ds(tok0, CHUNK)], ids_v.at[slot])
        pltpu.sync_copy(pos_hbm.at[pl.ds(tok0, CHUNK)], pos_v.at[slot])
        pltpu.async_copy(emb_hbm.at[ids_v.at[slot]], e_buf.at[slot],
                         g_sems[slot])

    def wait_gather(slot):
        pltpu.make_async_copy(emb_hbm.at[pl.ds(0, CHUNK)], e_buf.at[slot],
                              g_sems[slot]).wait()

    def wait_out(slot):
        pltpu.make_async_copy(s_buf.at[slot],
                              summed_hbm.at[pl.ds(w_base, CHUNK)],
                              o_sems[slot]).wait()

    def half(c2, c, slot):
        wait_gather(slot)

        @pl.when(c2 > 0)
        def _():
            wait_out(slot)

        def t_body(t, acc):
            tg = lax.shift_left(lax.shift_right_logical(t, 4), 4)
            lane = t - tg
            pv = pos_v[slot, pl.ds(tg, L)]
            base = lax.shift_left(_splat(pv, lane), 6) + iota
            new = []
            for j in range(NJ):
                e = e_buf[slot, t, pl.ds(j * L, L)]
                p = plsc.load_gather(pot_v, [base + (j * L)])
                s = e + p
                s_buf[slot, t, pl.ds(j * L, L)] = s
                new.append(acc[j] + s)
            return tuple(new)

        zeros = tuple(jnp.zeros((L,), jnp.float32) for _ in range(NJ))
        acc = lax.fori_loop(0, SEQ, t_body, zeros)
        for j in range(NJ):
            pool_buf[c, pl.ds(j * L, L)] = acc[j] * inv_seq

        pltpu.async_copy(s_buf.at[slot],
                         summed_hbm.at[pl.ds(tok0_of(c), CHUNK)],
                         o_sems[slot])

        @pl.when(c + 2 < NCHUNKS)
        def _():
            fetch(c + 2, slot)

    fetch(0, 0)
    fetch(1, 1)

    def pair_body(c2, carry):
        half(c2, 2 * c2, 0)
        half(c2, 2 * c2 + 1, 1)
        return carry

    lax.fori_loop(0, NCHUNKS // 2, pair_body, jnp.int32(0))

    wait_out(0)
    wait_out(1)
    pltpu.sync_copy(pool_buf, pooled_hbm.at[pl.ds(wid * ROWS_PER_W,
                                                  ROWS_PER_W)])


def kernel(input, positions, hidden, emb_table, pos_table):
    del hidden
    ids = input.reshape(BATCH * SEQ)
    pos = positions.reshape(BATCH * SEQ)
    pot = pos_table.reshape(SEQ * HIDDEN)
    summed_flat, pooled = _encoder_sc(ids, pos, emb_table, pot)
    return (pooled[None], summed_flat.reshape(BATCH, SEQ, HIDDEN))


# speedup vs baseline: 1.1066x; 1.1066x over previous
import functools

import jax
import jax.numpy as jnp
from jax import lax
from jax.experimental import pallas as pl
from jax.experimental.pallas import tpu as pltpu
from jax.experimental.pallas import tpu_sc as plsc

NC = 2
NS = 16
NW = NC * NS
L = 16

BATCH = 4096
SEQ = 200
HIDDEN = 64
NJ = HIDDEN // L

ROWS_PER_W = BATCH // NW
CHUNK = SEQ
NCHUNKS = ROWS_PER_W
TOK_PER_W = ROWS_PER_W * SEQ

SPLITS = ((0, 56), (56, 56), (112, 56), (168, 32))

_mesh = plsc.VectorSubcoreMesh(
    core_axis_name="c", subcore_axis_name="s", num_cores=NC, num_subcores=NS
)


@functools.partial(
    pl.kernel,
    out_type=(
        jax.ShapeDtypeStruct((BATCH * SEQ, HIDDEN), jnp.float32),
        jax.ShapeDtypeStruct((BATCH, HIDDEN), jnp.float32),
    ),
    mesh=_mesh,
    compiler_params=pltpu.CompilerParams(use_tc_tiling_on_sc=False),
    scratch_types=[
        pltpu.VMEM((2, CHUNK), jnp.int32),
        pltpu.VMEM((2, CHUNK), jnp.int32),
        pltpu.VMEM((2, CHUNK, HIDDEN), jnp.float32),
        pltpu.VMEM((2, CHUNK, HIDDEN), jnp.float32),
        pltpu.VMEM((2, CHUNK, HIDDEN), jnp.float32),
        pltpu.VMEM((ROWS_PER_W, HIDDEN), jnp.float32),
        pltpu.SemaphoreType.DMA,
        pltpu.SemaphoreType.DMA,
        pltpu.SemaphoreType.DMA,
        pltpu.SemaphoreType.DMA,
    ],
)
def _encoder_sc(ids_hbm, pos_hbm, emb_hbm, pot_hbm, summed_hbm, pooled_hbm,
                ids_v, pos_v, e_buf, p_buf, s_buf, pool_buf,
                g_sem0, g_sem1, o_sem0, o_sem1):
    wid = lax.axis_index("s") * NC + lax.axis_index("c")
    w_base = wid * TOK_PER_W
    inv_seq = jnp.float32(1.0 / SEQ)
    g_sems = (g_sem0, g_sem1)
    o_sems = (o_sem0, o_sem1)

    def tok0_of(c):
        return pl.multiple_of(w_base + c * CHUNK, CHUNK)

    def fetch(c, slot):
        tok0 = tok0_of(c)
        pltpu.sync_copy(ids_hbm.at[pl.ds(tok0, CHUNK)], ids_v.at[slot])
        pltpu.sync_copy(pos_hbm.at[pl.ds(tok0, CHUNK)], pos_v.at[slot])
        for off, n in SPLITS:
            pltpu.async_copy(
                emb_hbm.at[ids_v.at[slot].at[pl.ds(off, n)]],
                e_buf.at[slot].at[pl.ds(off, n)], g_sems[slot])
        for off, n in SPLITS:
            pltpu.async_copy(
                pot_hbm.at[pos_v.at[slot].at[pl.ds(off, n)]],
                p_buf.at[slot].at[pl.ds(off, n)], g_sems[slot])

    def wait_gathers(slot):
        pltpu.make_async_copy(emb_hbm.at[pl.ds(0, CHUNK)], e_buf.at[slot],
                              g_sems[slot]).wait()
        pltpu.make_async_copy(emb_hbm.at[pl.ds(0, CHUNK)], p_buf.at[slot],
                              g_sems[slot]).wait()

    def wait_out(slot):
        pltpu.make_async_copy(s_buf.at[slot],
                              summed_hbm.at[pl.ds(w_base, CHUNK)],
                              o_sems[slot]).wait()

    def half(c2, c, slot):
        wait_gathers(slot)

        @pl.when(c2 > 0)
        def _():
            wait_out(slot)

        def t_body(t, acc):
            new = []
            for j in range(NJ):
                e = e_buf[slot, t, pl.ds(j * L, L)]
                p = p_buf[slot, t, pl.ds(j * L, L)]
                s = e + p
                s_buf[slot, t, pl.ds(j * L, L)] = s
                new.append(acc[j] + s)
            return tuple(new)

        zeros = tuple(jnp.zeros((L,), jnp.float32) for _ in range(NJ))
        acc = lax.fori_loop(0, SEQ, t_body, zeros)
        for j in range(NJ):
            pool_buf[c, pl.ds(j * L, L)] = acc[j] * inv_seq

        pltpu.async_copy(s_buf.at[slot],
                         summed_hbm.at[pl.ds(tok0_of(c), CHUNK)],
                         o_sems[slot])

        @pl.when(c + 2 < NCHUNKS)
        def _():
            fetch(c + 2, slot)

    fetch(0, 0)
    fetch(1, 1)

    def pair_body(c2, carry):
        half(c2, 2 * c2, 0)
        half(c2, 2 * c2 + 1, 1)
        return carry

    lax.fori_loop(0, NCHUNKS // 2, pair_body, jnp.int32(0))

    wait_out(0)
    wait_out(1)
    pltpu.sync_copy(pool_buf, pooled_hbm.at[pl.ds(wid * ROWS_PER_W,
                                                  ROWS_PER_W)])


def kernel(input, positions, hidden, emb_table, pos_table):
    del hidden
    ids = input.reshape(BATCH * SEQ)
    pos = positions.reshape(BATCH * SEQ)
    summed_flat, pooled = _encoder_sc(ids, pos, emb_table, pos_table)
    return (pooled[None], summed_flat.reshape(BATCH, SEQ, HIDDEN))


# speedup vs baseline: 1.1083x; 1.0016x over previous
import functools

import jax
import jax.numpy as jnp
from jax import lax
from jax.experimental import pallas as pl
from jax.experimental.pallas import tpu as pltpu
from jax.experimental.pallas import tpu_sc as plsc

NC = 2
NS = 16
NW = NC * NS
L = 16

BATCH = 4096
SEQ = 200
HIDDEN = 64
NJ = HIDDEN // L

ROWS_PER_W = BATCH // NW
CHUNK = SEQ

_mesh = plsc.VectorSubcoreMesh(
    core_axis_name="c", subcore_axis_name="s", num_cores=NC, num_subcores=NS
)


@functools.partial(
    pl.kernel,
    out_type=(
        jax.ShapeDtypeStruct((BATCH, SEQ, HIDDEN), jnp.float32),
        jax.ShapeDtypeStruct((BATCH, HIDDEN), jnp.float32),
    ),
    mesh=_mesh,
    compiler_params=pltpu.CompilerParams(use_tc_tiling_on_sc=False),
    scratch_types=[
        pltpu.VMEM((2, CHUNK), jnp.int32),
        pltpu.VMEM((2, CHUNK), jnp.int32),
        pltpu.VMEM((2, CHUNK, HIDDEN), jnp.float32),
        pltpu.VMEM((2, CHUNK, HIDDEN), jnp.float32),
        pltpu.VMEM((2, CHUNK, HIDDEN), jnp.float32),
        pltpu.VMEM((ROWS_PER_W, HIDDEN), jnp.float32),
        pltpu.SemaphoreType.DMA,
        pltpu.SemaphoreType.DMA,
        pltpu.SemaphoreType.DMA,
        pltpu.SemaphoreType.DMA,
    ],
)
def _encoder_sc(ids_hbm, pos_hbm, emb_hbm, pot_hbm, summed_hbm, pooled_hbm,
                ids_v, pos_v, e_buf, p_buf, s_buf, pool_buf,
                g_sem0, g_sem1, o_sem0, o_sem1):
    wid = lax.axis_index("s") * NC + lax.axis_index("c")
    row0 = wid * ROWS_PER_W
    inv_seq = jnp.float32(1.0 / SEQ)
    g_sems = (g_sem0, g_sem1)
    o_sems = (o_sem0, o_sem1)

    def fetch(c, slot):
        b = row0 + c
        pltpu.sync_copy(ids_hbm.at[b], ids_v.at[slot])
        pltpu.sync_copy(pos_hbm.at[b], pos_v.at[slot])
        pltpu.async_copy(emb_hbm.at[ids_v.at[slot]], e_buf.at[slot],
                         g_sems[slot])
        pltpu.async_copy(pot_hbm.at[pos_v.at[slot]], p_buf.at[slot],
                         g_sems[slot])

    def wait_gathers(slot):
        pltpu.make_async_copy(pot_hbm, e_buf.at[slot], g_sems[slot]).wait()
        pltpu.make_async_copy(pot_hbm, p_buf.at[slot], g_sems[slot]).wait()

    def wait_out(slot):
        pltpu.make_async_copy(s_buf.at[slot], summed_hbm.at[row0],
                              o_sems[slot]).wait()

    def half(c2, c, slot):
        wait_gathers(slot)

        @pl.when(c2 > 0)
        def _():
            wait_out(slot)

        def t_body(t, acc):
            new = []
            for j in range(NJ):
                e = e_buf[slot, t, pl.ds(j * L, L)]
                p = p_buf[slot, t, pl.ds(j * L, L)]
                s = e + p
                s_buf[slot, t, pl.ds(j * L, L)] = s
                new.append(acc[j] + s)
            return tuple(new)

        zeros = tuple(jnp.zeros((L,), jnp.float32) for _ in range(NJ))
        acc = lax.fori_loop(0, SEQ, t_body, zeros)
        for j in range(NJ):
            pool_buf[c, pl.ds(j * L, L)] = acc[j] * inv_seq

        pltpu.async_copy(s_buf.at[slot], summed_hbm.at[row0 + c],
                         o_sems[slot])

        @pl.when(c + 2 < ROWS_PER_W)
        def _():
            fetch(c + 2, slot)

    def pair_body(c2, carry):
        half(c2, 2 * c2, 0)
        half(c2, 2 * c2 + 1, 1)
        return carry

    fetch(0, 0)
    fetch(1, 1)
    lax.fori_loop(0, ROWS_PER_W // 2, pair_body, jnp.int32(0))

    wait_out(0)
    wait_out(1)
    pltpu.sync_copy(pool_buf, pooled_hbm.at[pl.ds(row0, ROWS_PER_W)])


def kernel(input, positions, hidden, emb_table, pos_table):
    del hidden
    summed, pooled = _encoder_sc(input, positions, emb_table, pos_table)
    return (pooled[None], summed)


# speedup vs baseline: 1.1378x; 1.0266x over previous
import functools

import jax
import jax.numpy as jnp
from jax import lax
from jax.experimental import pallas as pl
from jax.experimental.pallas import tpu as pltpu
from jax.experimental.pallas import tpu_sc as plsc

NC = 2
NS = 16
NW = NC * NS
L = 16

BATCH = 4096
SEQ = 200
HIDDEN = 64
NJ = HIDDEN // L

ROWS_PER_W = BATCH // NW
CHUNK = SEQ
NG = SEQ // L
REM = SEQ - NG * L

_mesh = plsc.VectorSubcoreMesh(
    core_axis_name="c", subcore_axis_name="s", num_cores=NC, num_subcores=NS
)

_TAKE_DNUMS = lax.GatherDimensionNumbers(
    offset_dims=(), collapsed_slice_dims=(0,), start_index_map=(0,)
)


def _splat(vec, lane):
    idx = jnp.full((L,), lane, jnp.int32)
    return lax.gather(vec, idx[:, None], _TAKE_DNUMS, (1,),
                      mode=lax.GatherScatterMode.PROMISE_IN_BOUNDS)


@functools.partial(
    pl.kernel,
    out_type=(
        jax.ShapeDtypeStruct((BATCH, SEQ, HIDDEN), jnp.float32),
        jax.ShapeDtypeStruct((BATCH, HIDDEN), jnp.float32),
    ),
    mesh=_mesh,
    compiler_params=pltpu.CompilerParams(use_tc_tiling_on_sc=False,
                                         needs_layout_passes=False),
    scratch_types=[
        pltpu.VMEM((2, CHUNK), jnp.int32),
        pltpu.VMEM((2, CHUNK), jnp.int32),
        pltpu.VMEM((2, CHUNK, HIDDEN), jnp.float32),
        pltpu.VMEM((2, CHUNK, HIDDEN), jnp.float32),
        pltpu.VMEM((SEQ, HIDDEN), jnp.float32),
        pltpu.VMEM((ROWS_PER_W, HIDDEN), jnp.float32),
        pltpu.SemaphoreType.DMA,
        pltpu.SemaphoreType.DMA,
        pltpu.SemaphoreType.DMA,
        pltpu.SemaphoreType.DMA,
    ],
)
def _encoder_sc(ids_hbm, pos_hbm, emb_hbm, pot_hbm, summed_hbm, pooled_hbm,
                ids_v, pos_v, e_buf, s_buf, pot_v, pool_buf,
                g_sem0, g_sem1, o_sem0, o_sem1):
    wid = lax.axis_index("s") * NC + lax.axis_index("c")
    row0 = wid * ROWS_PER_W
    inv_seq = jnp.float32(1.0 / SEQ)
    g_sems = (g_sem0, g_sem1)
    o_sems = (o_sem0, o_sem1)
    cols = tuple(lax.iota(jnp.int32, L) + (j * L) for j in range(NJ))

    pltpu.sync_copy(pot_hbm, pot_v)

    def fetch(c, slot):
        b = row0 + c
        pltpu.sync_copy(ids_hbm.at[b], ids_v.at[slot])
        pltpu.sync_copy(pos_hbm.at[b], pos_v.at[slot])
        pltpu.async_copy(emb_hbm.at[ids_v.at[slot]], e_buf.at[slot],
                         g_sems[slot])

    def wait_gather(slot):
        pltpu.make_async_copy(emb_hbm.at[pl.ds(0, CHUNK)], e_buf.at[slot],
                              g_sems[slot]).wait()

    def wait_out(slot):
        pltpu.make_async_copy(s_buf.at[slot], summed_hbm.at[row0],
                              o_sems[slot]).wait()

    def half(c2, c, slot):
        wait_gather(slot)

        @pl.when(c2 > 0)
        def _():
            wait_out(slot)

        def tokens(tg, pv, lanes, acc):
            for i in lanes:
                row = tg + i
                prow = _splat(pv, i)
                for j in range(NJ):
                    e = e_buf[slot, row, pl.ds(j * L, L)]
                    p = plsc.load_gather(pot_v, [prow, cols[j]])
                    s = e + p
                    s_buf[slot, row, pl.ds(j * L, L)] = s
                    acc[j] = acc[j] + s
            return acc

        def g_body(g, acc):
            tg = pl.multiple_of(g * L, L)
            pv = pos_v[slot, pl.ds(tg, L)]
            return tuple(tokens(tg, pv, range(L), list(acc)))

        zeros = tuple(jnp.zeros((L,), jnp.float32) for _ in range(NJ))
        acc = lax.fori_loop(0, NG, g_body, zeros)
        tg = SEQ - L
        pv = pos_v[slot, pl.ds(tg, L)]
        acc = tokens(tg, pv, range(L - REM, L), list(acc))

        for j in range(NJ):
            pool_buf[c, pl.ds(j * L, L)] = acc[j] * inv_seq

        pltpu.async_copy(s_buf.at[slot], summed_hbm.at[row0 + c],
                         o_sems[slot])

        @pl.when(c + 2 < ROWS_PER_W)
        def _():
            fetch(c + 2, slot)

    def pair_body(c2, carry):
        half(c2, 2 * c2, 0)
        half(c2, 2 * c2 + 1, 1)
        return carry

    fetch(0, 0)
    fetch(1, 1)
    lax.fori_loop(0, ROWS_PER_W // 2, pair_body, jnp.int32(0))

    wait_out(0)
    wait_out(1)
    pltpu.sync_copy(pool_buf, pooled_hbm.at[pl.ds(row0, ROWS_PER_W)])


def kernel(input, positions, hidden, emb_table, pos_table):
    del hidden
    summed, pooled = _encoder_sc(input, positions, emb_table, pos_table)
    return (pooled[None], summed)


# speedup vs baseline: 1.2292x; 1.0803x over previous
import functools

import jax
import jax.numpy as jnp
from jax import lax
from jax.experimental import pallas as pl
from jax.experimental.pallas import tpu as pltpu
from jax.experimental.pallas import tpu_sc as plsc

NC = 2
NS = 16
NW = NC * NS
L = 16

BATCH = 4096
SEQ = 200
HIDDEN = 64
NJ = HIDDEN // L

ROWS_PER_W = BATCH // NW
CHUNK = SEQ
NG = SEQ // L
REM = SEQ - NG * L

_mesh = plsc.VectorSubcoreMesh(
    core_axis_name="c", subcore_axis_name="s", num_cores=NC, num_subcores=NS
)

_TAKE_DNUMS = lax.GatherDimensionNumbers(
    offset_dims=(), collapsed_slice_dims=(0,), start_index_map=(0,)
)


def _splat(vec, lane):
    idx = jnp.full((L,), lane, jnp.int32)
    return lax.gather(vec, idx[:, None], _TAKE_DNUMS, (1,),
                      mode=lax.GatherScatterMode.PROMISE_IN_BOUNDS)


@functools.partial(
    pl.kernel,
    out_type=(
        jax.ShapeDtypeStruct((BATCH, SEQ, HIDDEN), jnp.float32),
        jax.ShapeDtypeStruct((BATCH, HIDDEN), jnp.float32),
    ),
    mesh=_mesh,
    compiler_params=pltpu.CompilerParams(use_tc_tiling_on_sc=False,
                                         needs_layout_passes=False),
    scratch_types=[
        pltpu.VMEM((ROWS_PER_W, CHUNK), jnp.int32),
        pltpu.VMEM((ROWS_PER_W, CHUNK), jnp.int32),
        pltpu.VMEM((2, CHUNK, HIDDEN), jnp.float32),
        pltpu.VMEM((2, CHUNK, HIDDEN), jnp.float32),
        pltpu.VMEM((SEQ, HIDDEN), jnp.float32),
        pltpu.VMEM((ROWS_PER_W, HIDDEN), jnp.float32),
        pltpu.SemaphoreType.DMA,
        pltpu.SemaphoreType.DMA,
        pltpu.SemaphoreType.DMA,
        pltpu.SemaphoreType.DMA,
    ],
)
def _encoder_sc(ids_hbm, pos_hbm, emb_hbm, pot_hbm, summed_hbm, pooled_hbm,
                ids_v, pos_v, e_buf, s_buf, pot_v, pool_buf,
                g_sem0, g_sem1, o_sem0, o_sem1):
    wid = lax.axis_index("s") * NC + lax.axis_index("c")
    row0 = wid * ROWS_PER_W
    inv_seq = jnp.float32(1.0 / SEQ)
    g_sems = (g_sem0, g_sem1)
    o_sems = (o_sem0, o_sem1)
    cols = tuple(lax.iota(jnp.int32, L) + (j * L) for j in range(NJ))

    pltpu.sync_copy(pot_hbm, pot_v)
    pltpu.sync_copy(ids_hbm.at[pl.ds(row0, ROWS_PER_W)], ids_v)
    pltpu.sync_copy(pos_hbm.at[pl.ds(row0, ROWS_PER_W)], pos_v)

    def fetch(c, slot):
        pltpu.async_copy(emb_hbm.at[ids_v.at[c]], e_buf.at[slot],
                         g_sems[slot])

    def wait_gather(slot):
        pltpu.make_async_copy(emb_hbm.at[pl.ds(0, CHUNK)], e_buf.at[slot],
                              g_sems[slot]).wait()

    def wait_out(slot):
        pltpu.make_async_copy(s_buf.at[slot], summed_hbm.at[row0],
                              o_sems[slot]).wait()

    def half(c2, c, slot):
        wait_gather(slot)

        @pl.when(c2 > 0)
        def _():
            wait_out(slot)

        def tokens(tg, pv, lanes, acc):
            for i in lanes:
                row = tg + i
                prow = _splat(pv, i)
                for j in range(NJ):
                    e = e_buf[slot, row, pl.ds(j * L, L)]
                    p = plsc.load_gather(pot_v, [prow, cols[j]])
                    s = e + p
                    s_buf[slot, row, pl.ds(j * L, L)] = s
                    acc[j] = acc[j] + s
            return acc

        def g_body(g, acc):
            tg = pl.multiple_of(g * L, L)
            pv = pos_v[c, pl.ds(tg, L)]
            return tuple(tokens(tg, pv, range(L), list(acc)))

        zeros = tuple(jnp.zeros((L,), jnp.float32) for _ in range(NJ))
        acc = lax.fori_loop(0, NG, g_body, zeros)
        tg = SEQ - L
        pv = pos_v[c, pl.ds(tg, L)]
        acc = tokens(tg, pv, range(L - REM, L), list(acc))

        for j in range(NJ):
            pool_buf[c, pl.ds(j * L, L)] = acc[j] * inv_seq

        pltpu.async_copy(s_buf.at[slot], summed_hbm.at[row0 + c],
                         o_sems[slot])

        @pl.when(c + 2 < ROWS_PER_W)
        def _():
            fetch(c + 2, slot)

    def pair_body(c2, carry):
        half(c2, 2 * c2, 0)
        half(c2, 2 * c2 + 1, 1)
        return carry

    fetch(0, 0)
    fetch(1, 1)
    lax.fori_loop(0, ROWS_PER_W // 2, pair_body, jnp.int32(0))

    wait_out(0)
    wait_out(1)
    pltpu.sync_copy(pool_buf, pooled_hbm.at[pl.ds(row0, ROWS_PER_W)])


def kernel(input, positions, hidden, emb_table, pos_table):
    del hidden
    summed, pooled = _encoder_sc(input, positions, emb_table, pos_table)
    return (pooled[None], summed)
